# baseline (speedup 1.0000x reference)
import numpy as np
import jax
import jax.numpy as jnp
from jax.experimental import pallas as pl
from jax.experimental.pallas import tpu as pltpu

N_BINS = 15
_BOUNDS = np.linspace(0.0, 1.0, N_BINS + 1)


def _ece_tc_kernel(n_total, logits_ref, labels_ref, bounds_ref, cnt_ref,
                   sc_ref, sa_ref, ece_ref):
    i = pl.program_id(0)
    nsteps = pl.num_programs(0)

    x = logits_ref[...]
    c = x.shape[1]
    m = jnp.max(x, axis=1, keepdims=True)
    s = jnp.sum(jnp.exp(x - m), axis=1, keepdims=True)
    conf = 1.0 / s
    iota = jax.lax.broadcasted_iota(jnp.int32, x.shape, 1)
    pred = jnp.min(jnp.where(x == m, iota, c), axis=1, keepdims=True)
    acc = (pred == labels_ref[...]).astype(jnp.float32)

    lo = bounds_ref[0:1, :]
    hi = bounds_ref[1:2, :]
    inb = ((conf > lo) & (conf <= hi)).astype(jnp.float32)
    cnt_p = jnp.sum(inb, axis=0, keepdims=True)
    sc_p = jnp.sum(inb * conf, axis=0, keepdims=True)
    sa_p = jnp.sum(inb * acc, axis=0, keepdims=True)

    @pl.when(i == 0)
    def _init():
        cnt_ref[...] = jnp.zeros_like(cnt_ref)
        sc_ref[...] = jnp.zeros_like(sc_ref)
        sa_ref[...] = jnp.zeros_like(sa_ref)

    cnt_ref[...] += cnt_p
    sc_ref[...] += sc_p
    sa_ref[...] += sa_p

    @pl.when(i == nsteps - 1)
    def _finish():
        cnt = cnt_ref[...]
        safe = jnp.maximum(cnt, 1.0)
        avg_conf = sc_ref[...] / safe
        avg_acc = sa_ref[...] / safe
        prop = cnt / np.float32(n_total)
        contrib = jnp.abs(avg_conf - avg_acc) * prop
        ece_ref[...] = jnp.sum(jnp.where(cnt > 0, contrib, 0.0),
                               keepdims=True)


def kernel(logits, labels):
    n, c = logits.shape
    block = 8000
    assert n % block == 0
    nsteps = n // block
    labels2 = labels.reshape(n, 1)
    bounds = jnp.asarray(
        np.stack([_BOUNDS[:-1], _BOUNDS[1:]]).astype(np.float32))

    import functools
    body = functools.partial(_ece_tc_kernel, n)
    out = pl.pallas_call(
        body,
        grid=(nsteps,),
        in_specs=[
            pl.BlockSpec((block, c), lambda i: (i, 0)),
            pl.BlockSpec((block, 1), lambda i: (i, 0)),
            pl.BlockSpec((2, N_BINS), lambda i: (0, 0)),
        ],
        out_specs=[
            pl.BlockSpec((1, N_BINS), lambda i: (0, 0)),
            pl.BlockSpec((1, N_BINS), lambda i: (0, 0)),
            pl.BlockSpec((1, N_BINS), lambda i: (0, 0)),
            pl.BlockSpec((1, 1), lambda i: (0, 0)),
        ],
        out_shape=[
            jax.ShapeDtypeStruct((1, N_BINS), jnp.float32),
            jax.ShapeDtypeStruct((1, N_BINS), jnp.float32),
            jax.ShapeDtypeStruct((1, N_BINS), jnp.float32),
            jax.ShapeDtypeStruct((1, 1), jnp.float32),
        ],
        compiler_params=pltpu.CompilerParams(
            dimension_semantics=("arbitrary",),
        ),
    )(logits, labels2, bounds)
    return out[3].reshape(1)
